# baseline (speedup 1.0000x reference)
import functools

import jax
import jax.numpy as jnp
from jax import lax
from jax.experimental import pallas as pl
from jax.experimental.pallas import tpu as pltpu
from jax.experimental.pallas import tpu_sc as plsc

VOCAB = 1000000
EMBED_DIM = 32
BATCH = 16384
N_FIELDS = 26

_B = BATCH * N_FIELDS
_LINE = 128
_CHUNK = 32
_LINES = VOCAB // 4


def _make_kernel():
    info = plsc.get_sparse_core_info()
    nc, ns = info.num_cores, info.num_subcores
    nw = nc * ns
    b_pw = _B // nw
    n_chunks = b_pw // _CHUNK
    n_pairs = n_chunks // 2

    mesh = plsc.VectorSubcoreMesh(core_axis_name="c", subcore_axis_name="s")

    @functools.partial(
        pl.kernel,
        mesh=mesh,
        out_type=jax.ShapeDtypeStruct((_B, _LINE), jnp.float32),
        scratch_types=[
            pltpu.VMEM((b_pw,), jnp.int32),
            pltpu.VMEM((b_pw,), jnp.int32),
            pltpu.VMEM((2 * _CHUNK, _LINE), jnp.float32),
            pltpu.SemaphoreType.DMA,
            pltpu.SemaphoreType.DMA,
        ],
    )
    def emb_kernel(idx_hbm, table_hbm, out_hbm,
                   idx_v, lidx_v, lines_v, sem0, sem1):
        wid = lax.axis_index("s") * nc + lax.axis_index("c")
        b0 = wid * b_pw

        pltpu.sync_copy(idx_hbm.at[pl.ds(b0, b_pw)], idx_v)

        def clamp_body(r, _):
            for c in range(8):
                v = idx_v[pl.ds(r * 128 + c * 16, 16)]
                cv = jnp.maximum(v - 1, 0)
                lidx_v[pl.ds(r * 128 + c * 16, 16)] = (
                    lax.shift_right_logical(cv, 2))
            return ()
        lax.fori_loop(0, b_pw // 128, clamp_body, ())

        sems = (sem0, sem1)

        def gather(ch, slot):
            return pltpu.async_copy(
                table_hbm.at[lidx_v.at[pl.ds(ch * _CHUNK, _CHUNK)]],
                lines_v.at[pl.ds(slot * _CHUNK, _CHUNK)],
                sems[slot],
            )

        def wait(ch, slot):
            pltpu.make_async_copy(
                table_hbm.at[lidx_v.at[pl.ds(ch * _CHUNK, _CHUNK)]],
                lines_v.at[pl.ds(slot * _CHUNK, _CHUNK)],
                sems[slot],
            ).wait()

        def writeback(ch, slot):
            pltpu.sync_copy(
                lines_v.at[pl.ds(slot * _CHUNK, _CHUNK)],
                out_hbm.at[pl.ds(b0 + ch * _CHUNK, _CHUNK)],
            )

        gather(0, 0)

        def pair_body(g, _):
            ch = g * 2
            wait(ch, 0)
            gather(ch + 1, 1)
            writeback(ch, 0)
            wait(ch + 1, 1)

            @pl.when(g + 1 < n_pairs)
            def _next():
                gather(ch + 2, 0)

            writeback(ch + 1, 1)
            return ()

        lax.fori_loop(0, n_pairs, pair_body, ())

    return emb_kernel


def kernel(q_idx, embed_para):
    idx_flat = q_idx.astype(jnp.int32).reshape(-1)
    table_lines = embed_para.reshape(_LINES, _LINE)
    lines = _make_kernel()(idx_flat, table_lines)
    cv = jnp.maximum(idx_flat - 1, 0)
    onehot = (cv & 3)[:, None] == jnp.arange(4, dtype=jnp.int32)[None, :]
    onehot = jnp.where((idx_flat != 0)[:, None], onehot, False)
    out = jnp.einsum(
        "bq,bqd->bd",
        onehot.astype(jnp.float32),
        lines.reshape(_B, 4, EMBED_DIM),
        precision=jax.lax.Precision.DEFAULT,
    )
    return out.reshape(BATCH, N_FIELDS, EMBED_DIM)


# speedup vs baseline: 1.6206x; 1.6206x over previous
import functools

import jax
import jax.numpy as jnp
from jax import lax
from jax.experimental import pallas as pl
from jax.experimental.pallas import tpu as pltpu
from jax.experimental.pallas import tpu_sc as plsc

VOCAB = 1000000
EMBED_DIM = 32
BATCH = 16384
N_FIELDS = 26

_B = BATCH * N_FIELDS
_LINE = 128
_LINES = VOCAB // 4
_GRP = 208
_C0 = 112
_C1 = _GRP - _C0


def _make_kernel():
    info = plsc.get_sparse_core_info()
    nc, ns = info.num_cores, info.num_subcores
    nw = nc * ns
    b_pw = _B // nw
    n_groups = b_pw // _GRP
    n_gpairs = n_groups // 2

    mesh = plsc.VectorSubcoreMesh(core_axis_name="c", subcore_axis_name="s")

    @functools.partial(
        pl.kernel,
        mesh=mesh,
        out_type=jax.ShapeDtypeStruct((_B, EMBED_DIM), jnp.float32),
        scratch_types=[
            pltpu.VMEM((b_pw,), jnp.int32),
            pltpu.VMEM((b_pw,), jnp.int32),
            pltpu.VMEM((_C0, _LINE), jnp.float32),
            pltpu.VMEM((_C1, _LINE), jnp.float32),
            pltpu.VMEM((_GRP, EMBED_DIM), jnp.float32),
            pltpu.VMEM((_GRP, EMBED_DIM), jnp.float32),
            pltpu.SemaphoreType.DMA,
            pltpu.SemaphoreType.DMA,
            pltpu.SemaphoreType.DMA,
            pltpu.SemaphoreType.DMA,
        ],
    )
    def emb_kernel(idx_hbm, table_hbm, out_hbm,
                   idx_v, lidx_v, ln0_v, ln1_v, ob0_v, ob1_v,
                   sg0, sg1, sw0, sw1):
        wid = lax.axis_index("s") * nc + lax.axis_index("c")
        b0 = wid * b_pw

        pltpu.sync_copy(idx_hbm.at[pl.ds(b0, b_pw)], idx_v)

        def clamp_body(r, _):
            for c in range(8):
                v = idx_v[pl.ds(r * 128 + c * 16, 16)]
                cv = jnp.maximum(v - 1, 0)
                lidx_v[pl.ds(r * 128 + c * 16, 16)] = (
                    lax.shift_right_logical(cv, 2))
            return ()
        lax.fori_loop(0, b_pw // 128, clamp_body, ())

        lnbufs = (ln0_v, ln1_v)
        gsems = (sg0, sg1)
        wsems = (sw0, sw1)
        obufs = (ob0_v, ob1_v)
        chunk_of = ((0, _C0), (_C0, _C1))

        def gather(g, slot):
            off, n = chunk_of[slot]
            return pltpu.async_copy(
                table_hbm.at[lidx_v.at[pl.ds(g * _GRP + off, n)]],
                lnbufs[slot],
                gsems[slot],
            )

        def gwait(g, slot):
            off, n = chunk_of[slot]
            pltpu.make_async_copy(
                table_hbm.at[lidx_v.at[pl.ds(g * _GRP + off, n)]],
                lnbufs[slot],
                gsems[slot],
            ).wait()

        def wb(g, p):
            return pltpu.async_copy(
                obufs[p],
                out_hbm.at[pl.ds(b0 + g * _GRP, _GRP)],
                wsems[p],
            )

        def wb_wait(g, p):
            pltpu.make_async_copy(
                obufs[p],
                out_hbm.at[pl.ds(b0 + g * _GRP, _GRP)],
                wsems[p],
            ).wait()

        def select_chunk(g, slot, p):
            off, n = chunk_of[slot]
            lines = lnbufs[slot]
            obuf = obufs[p]
            for k in range(n // 16):
                pos = g * _GRP + off + k * 16
                v = idx_v[pl.ds(pos, 16)]
                cv = jnp.maximum(v - 1, 0)
                qv = (cv & 3) * EMBED_DIM
                keepf = jnp.minimum(v, 1).astype(jnp.float32)
                for l in range(16):
                    row = k * 16 + l
                    start = qv[l]
                    kf = keepf[l]
                    lo = lines[row, pl.ds(start, 16)] * kf
                    hi = lines[row, pl.ds(start + 16, 16)] * kf
                    obuf[off + row, pl.ds(0, 16)] = lo
                    obuf[off + row, pl.ds(16, 16)] = hi

        gather(0, 0)

        def pair_body(gp, _):
            for p in range(2):
                g = gp * 2 + p

                @pl.when(gp >= 1)
                def _drain(g=g, p=p):
                    wb_wait(g - 2, p)

                gwait(g, 0)
                gather(g, 1)
                select_chunk(g, 0, p)
                gwait(g, 1)

                @pl.when(g + 1 < n_groups)
                def _nxt(g=g):
                    gather(g + 1, 0)

                select_chunk(g, 1, p)
                wb(g, p)
            return ()

        lax.fori_loop(0, n_gpairs, pair_body, ())

        wb_wait(n_groups - 2, 0)
        wb_wait(n_groups - 1, 1)

    return emb_kernel


def kernel(q_idx, embed_para):
    idx_flat = q_idx.astype(jnp.int32).reshape(-1)
    table_lines = embed_para.reshape(_LINES, _LINE)
    out = _make_kernel()(idx_flat, table_lines)
    return out.reshape(BATCH, N_FIELDS, EMBED_DIM)


# speedup vs baseline: 1.8354x; 1.1326x over previous
import functools

import jax
import jax.numpy as jnp
from jax import lax
from jax.experimental import pallas as pl
from jax.experimental.pallas import tpu as pltpu
from jax.experimental.pallas import tpu_sc as plsc

VOCAB = 1000000
EMBED_DIM = 32
BATCH = 16384
N_FIELDS = 26

_B = BATCH * N_FIELDS
_LINE = 128
_LINES = VOCAB // 4
_GRP = 208
_C0 = 112
_C1 = _GRP - _C0


def _make_kernel():
    info = plsc.get_sparse_core_info()
    nc, ns = info.num_cores, info.num_subcores
    nw = nc * ns
    b_pw = _B // nw
    n_groups = b_pw // _GRP
    n_gpairs = n_groups // 2

    mesh = plsc.VectorSubcoreMesh(core_axis_name="c", subcore_axis_name="s")

    @functools.partial(
        pl.kernel,
        mesh=mesh,
        out_type=jax.ShapeDtypeStruct((BATCH, N_FIELDS, EMBED_DIM), jnp.float32),
        scratch_types=[
            pltpu.VMEM((b_pw,), jnp.int32),
            pltpu.VMEM((b_pw,), jnp.int32),
            pltpu.VMEM((_C0, _LINE), jnp.float32),
            pltpu.VMEM((_C1, _LINE), jnp.float32),
            pltpu.VMEM((_GRP, EMBED_DIM), jnp.float32),
            pltpu.VMEM((_GRP, EMBED_DIM), jnp.float32),
            pltpu.SemaphoreType.DMA,
            pltpu.SemaphoreType.DMA,
            pltpu.SemaphoreType.DMA,
            pltpu.SemaphoreType.DMA,
        ],
    )
    def emb_kernel(idx_hbm, table_hbm, out_hbm,
                   idx_v, lidx_v, ln0_v, ln1_v, ob0_v, ob1_v,
                   sg0, sg1, sw0, sw1):
        wid = lax.axis_index("s") * nc + lax.axis_index("c")
        b0 = wid * b_pw

        pltpu.sync_copy(idx_hbm.at[pl.ds(b0, b_pw)], idx_v)

        def clamp_body(r, _):
            for c in range(8):
                v = idx_v[pl.ds(r * 128 + c * 16, 16)]
                cv = jnp.maximum(v - 1, 0)
                lidx_v[pl.ds(r * 128 + c * 16, 16)] = (
                    lax.shift_right_logical(cv, 2))
            return ()
        lax.fori_loop(0, b_pw // 128, clamp_body, ())

        lnbufs = (ln0_v, ln1_v)
        gsems = (sg0, sg1)
        wsems = (sw0, sw1)
        obufs = (ob0_v, ob1_v)
        chunk_of = ((0, _C0), (_C0, _C1))

        def gather(g, slot):
            off, n = chunk_of[slot]
            return pltpu.async_copy(
                table_hbm.at[lidx_v.at[pl.ds(g * _GRP + off, n)]],
                lnbufs[slot],
                gsems[slot],
            )

        def gwait(g, slot):
            off, n = chunk_of[slot]
            pltpu.make_async_copy(
                table_hbm.at[lidx_v.at[pl.ds(g * _GRP + off, n)]],
                lnbufs[slot],
                gsems[slot],
            ).wait()

        item0 = wid * (b_pw // N_FIELDS)

        def wb(g, p):
            for i in range(_GRP // N_FIELDS):
                pltpu.async_copy(
                    obufs[p].at[pl.ds(i * N_FIELDS, N_FIELDS)],
                    out_hbm.at[item0 + g * (_GRP // N_FIELDS) + i],
                    wsems[p],
                )

        def wb_wait(g, p):
            for i in range(_GRP // N_FIELDS):
                pltpu.make_async_copy(
                    obufs[p].at[pl.ds(i * N_FIELDS, N_FIELDS)],
                    out_hbm.at[item0 + g * (_GRP // N_FIELDS) + i],
                    wsems[p],
                ).wait()

        def select_chunk(g, slot, p):
            off, n = chunk_of[slot]
            lines = lnbufs[slot]
            obuf = obufs[p]
            for k in range(n // 16):
                pos = g * _GRP + off + k * 16
                v = idx_v[pl.ds(pos, 16)]
                cv = jnp.maximum(v - 1, 0)
                qv = (cv & 3) * EMBED_DIM
                keepf = jnp.minimum(v, 1).astype(jnp.float32)
                for l in range(16):
                    row = k * 16 + l
                    start = qv[l]
                    kf = keepf[l]
                    lo = lines[row, pl.ds(start, 16)] * kf
                    hi = lines[row, pl.ds(start + 16, 16)] * kf
                    obuf[off + row, pl.ds(0, 16)] = lo
                    obuf[off + row, pl.ds(16, 16)] = hi

        gather(0, 0)

        def pair_body(gp, _):
            for p in range(2):
                g = gp * 2 + p

                @pl.when(gp >= 1)
                def _drain(g=g, p=p):
                    wb_wait(g - 2, p)

                gwait(g, 0)
                gather(g, 1)
                select_chunk(g, 0, p)
                gwait(g, 1)

                @pl.when(g + 1 < n_groups)
                def _nxt(g=g):
                    gather(g + 1, 0)

                select_chunk(g, 1, p)
                wb(g, p)
            return ()

        lax.fori_loop(0, n_gpairs, pair_body, ())

        wb_wait(n_groups - 2, 0)
        wb_wait(n_groups - 1, 1)

    return emb_kernel


def kernel(q_idx, embed_para):
    idx_flat = q_idx.astype(jnp.int32).reshape(-1)
    table_lines = embed_para.reshape(_LINES, _LINE)
    return _make_kernel()(idx_flat, table_lines)
